# Initial kernel scaffold; baseline (speedup 1.0000x reference)
import jax
import jax.numpy as jnp
from jax.experimental import pallas as pl


def kernel(x, Wg, W1, b1, W2, b2):
    raise NotImplementedError("write your pallas kernel here")


# speedup vs baseline: 2.7919x; 2.7919x over previous
import functools
import math

import jax
import jax.numpy as jnp
from jax import lax
from jax.experimental import pallas as pl
from jax.experimental.pallas import tpu as pltpu
from jax.experimental.pallas import tpu_sc as plsc

SC_CORES = 2
SC_SUBCORES = 16
NW = SC_CORES * SC_SUBCORES
SC_CHUNK = 64


def _rank_body(s_col_ref, s_row_ref, e_col_ref, e_row_ref, dest_ref, *,
               bt, cap, nslot):
    ch = 1024
    nch = bt // ch
    for ic in range(nch):
        si = s_col_ref[ic * ch:(ic + 1) * ch, :]
        ei = e_col_ref[ic * ch:(ic + 1) * ch, :]
        cnt = jnp.zeros((ch, 1), jnp.int32)
        for jc in range(nch):
            sj = s_row_ref[:, jc * ch:(jc + 1) * ch]
            ej = e_row_ref[:, jc * ch:(jc + 1) * ch]
            jg = jc * ch + lax.broadcasted_iota(jnp.int32, (ch, ch), 1)
            ig = ic * ch + lax.broadcasted_iota(jnp.int32, (ch, ch), 0)
            same = ej == ei
            beat = (sj > si) | ((sj == si) & (jg < ig))
            m = (same & beat).astype(jnp.int32)
            cnt = cnt + jnp.sum(m, axis=1, keepdims=True)
        iv = ic * ch + lax.broadcasted_iota(jnp.int32, (ch, 1), 0)
        dest = jnp.where(cnt < cap, ei * cap + cnt, nslot + iv)
        dest_ref[ic * ch:(ic + 1) * ch, :] = dest


def _ffn_body(z_ref, w1_ref, b1_ref, w2_ref, b2_ref, out_ref):
    h = pl.program_id(1)
    xb = z_ref[...].astype(jnp.bfloat16)
    w1 = w1_ref[0].astype(jnp.bfloat16)
    hid = jnp.dot(xb, w1, preferred_element_type=jnp.float32) + b1_ref[0]
    hid = jnp.maximum(hid, 0.0).astype(jnp.bfloat16)
    w2 = w2_ref[0].astype(jnp.bfloat16)
    contrib = jnp.dot(hid, w2, preferred_element_type=jnp.float32)

    @pl.when(h == 0)
    def _():
        out_ref[...] = b2_ref[0] + contrib

    @pl.when(h != 0)
    def _():
        out_ref[...] = out_ref[...] + contrib


def _make_dispatch(bt, d, zrows):
    iters = bt // (NW * SC_CHUNK)
    mesh = plsc.VectorSubcoreMesh(core_axis_name="c", subcore_axis_name="s")

    @functools.partial(
        pl.kernel,
        out_type=jax.ShapeDtypeStruct((zrows, d), jnp.float32),
        mesh=mesh,
        scratch_types=[
            pltpu.VMEM((SC_CHUNK,), jnp.int32),
            pltpu.VMEM((SC_CHUNK, d), jnp.float32),
            pltpu.SemaphoreType.DMA,
        ],
    )
    def dispatch(x_hbm, dest_hbm, z_hbm, idx_v, rows_v, sem):
        wid = lax.axis_index("s") * SC_CORES + lax.axis_index("c")
        for c in range(iters):
            base = wid * (iters * SC_CHUNK) + c * SC_CHUNK
            pltpu.sync_copy(dest_hbm.at[pl.ds(base, SC_CHUNK)], idx_v)
            pltpu.sync_copy(x_hbm.at[pl.ds(base, SC_CHUNK)], rows_v)
            pltpu.async_copy(rows_v, z_hbm.at[idx_v], sem).wait()

    return dispatch


def _make_combine(bt, d):
    iters = bt // (NW * SC_CHUNK)
    mesh = plsc.VectorSubcoreMesh(core_axis_name="c", subcore_axis_name="s")

    @functools.partial(
        pl.kernel,
        out_type=jax.ShapeDtypeStruct((bt, d), jnp.float32),
        mesh=mesh,
        scratch_types=[
            pltpu.VMEM((SC_CHUNK,), jnp.int32),
            pltpu.VMEM((SC_CHUNK, d), jnp.float32),
            pltpu.SemaphoreType.DMA,
        ],
    )
    def combine(big_hbm, dest_hbm, y_hbm, idx_v, rows_v, sem):
        wid = lax.axis_index("s") * SC_CORES + lax.axis_index("c")
        for c in range(iters):
            base = wid * (iters * SC_CHUNK) + c * SC_CHUNK
            pltpu.sync_copy(dest_hbm.at[pl.ds(base, SC_CHUNK)], idx_v)
            pltpu.async_copy(big_hbm.at[idx_v], rows_v, sem).wait()
            pltpu.sync_copy(rows_v, y_hbm.at[pl.ds(base, SC_CHUNK)])

    return combine


def kernel(x, Wg, W1, b1, W2, b2):
    bb, tt, d = x.shape
    n = Wg.shape[1]
    hdim = W1.shape[2]
    bt = bb * tt
    cap = math.ceil(1.25 * (bt / n))
    nslot = n * cap
    zrows = nslot + bt
    hb = 1024
    nh = hdim // hb

    logits = jnp.einsum('btd,dn->btn', x, Wg)
    probs = jax.nn.softmax(logits, axis=-1)
    pf = probs.reshape(bt, n)
    top1_val = jnp.max(pf, axis=-1)
    top1_idx = jnp.argmax(pf, axis=-1).astype(jnp.int32)
    x_flat = x.reshape(bt, d)

    dest_col = pl.pallas_call(
        functools.partial(_rank_body, bt=bt, cap=cap, nslot=nslot),
        out_shape=jax.ShapeDtypeStruct((bt, 1), jnp.int32),
    )(top1_val.reshape(bt, 1), top1_val.reshape(1, bt),
      top1_idx.reshape(bt, 1), top1_idx.reshape(1, bt))
    dest = dest_col.reshape(bt)

    z = _make_dispatch(bt, d, zrows)(x_flat, dest)

    big = pl.pallas_call(
        _ffn_body,
        grid=(n, nh),
        in_specs=[
            pl.BlockSpec((cap, d), lambda e, h: (e, 0)),
            pl.BlockSpec((1, d, hb), lambda e, h: (e, 0, h)),
            pl.BlockSpec((1, 1, hb), lambda e, h: (e, 0, h)),
            pl.BlockSpec((1, hb, d), lambda e, h: (e, h, 0)),
            pl.BlockSpec((1, 1, d), lambda e, h: (e, 0, 0)),
        ],
        out_specs=pl.BlockSpec((cap, d), lambda e, h: (e, 0)),
        out_shape=jax.ShapeDtypeStruct((zrows, d), jnp.float32),
        input_output_aliases={0: 0},
    )(z, W1, b1.reshape(n, 1, hdim), W2, b2.reshape(n, 1, d))

    y = _make_combine(bt, d)(big, dest)
    return y.reshape(bb, tt, d)


# speedup vs baseline: 3.3409x; 1.1967x over previous
import functools
import math

import jax
import jax.numpy as jnp
from jax import lax
from jax.experimental import pallas as pl
from jax.experimental.pallas import tpu as pltpu
from jax.experimental.pallas import tpu_sc as plsc

SC_CORES = 2
SC_SUBCORES = 16
NW = SC_CORES * SC_SUBCORES
SC_CHUNK = 64


def _rank_body(s_ref, e_ref, dest_ref, *, bt, n, cap, nslot):
    ch = 1024
    nch = bt // ch
    srow = s_ref[...]
    erow = e_ref[...]
    krow = erow * 67108864 + (lax.bitcast_convert_type(srow, jnp.int32)
                              - 1040187392)
    hists = [jnp.sum((erow == e).astype(jnp.int32)) for e in range(n)]
    jlt = (lax.broadcasted_iota(jnp.int32, (ch, ch), 0)
           < lax.broadcasted_iota(jnp.int32, (ch, ch), 1))
    for ic in range(nch):
        ki = krow[:, ic * ch:(ic + 1) * ch]
        ei = erow[:, ic * ch:(ic + 1) * ch]
        cnt = jnp.zeros((1, ch), jnp.int32)
        for jc in range(nch):
            kj = jnp.transpose(krow[:, jc * ch:(jc + 1) * ch])
            if jc < ic:
                m = kj >= ki
            elif jc > ic:
                m = kj > ki
            else:
                m = (kj > ki) | ((kj == ki) & jlt)
            cnt = cnt + jnp.sum(m.astype(jnp.int32), axis=0, keepdims=True)
        hc = jnp.zeros((1, ch), jnp.int32)
        for e in range(1, n):
            hc = hc + jnp.where(ei < e, hists[e], 0)
        rank = cnt - hc
        iv = ic * ch + lax.broadcasted_iota(jnp.int32, (1, ch), 1)
        dest_ref[:, ic * ch:(ic + 1) * ch] = jnp.where(
            rank < cap, ei * cap + rank, nslot + iv)


def _ffn_body(z_ref, w1_ref, b1_ref, w2_ref, b2_ref, out_ref):
    h = pl.program_id(1)
    xb = z_ref[...].astype(jnp.bfloat16)
    w1 = w1_ref[0].astype(jnp.bfloat16)
    hid = jnp.dot(xb, w1, preferred_element_type=jnp.float32) + b1_ref[0]
    hid = jnp.maximum(hid, 0.0).astype(jnp.bfloat16)
    w2 = w2_ref[0].astype(jnp.bfloat16)
    contrib = jnp.dot(hid, w2, preferred_element_type=jnp.float32)

    @pl.when(h == 0)
    def _():
        out_ref[...] = b2_ref[0] + contrib

    @pl.when(h != 0)
    def _():
        out_ref[...] = out_ref[...] + contrib


def _make_dispatch(bt, d, zrows):
    iters = bt // (NW * SC_CHUNK)
    mesh = plsc.VectorSubcoreMesh(core_axis_name="c", subcore_axis_name="s")

    @functools.partial(
        pl.kernel,
        out_type=jax.ShapeDtypeStruct((zrows, d), jnp.float32),
        mesh=mesh,
        scratch_types=[
            pltpu.VMEM((SC_CHUNK,), jnp.int32),
            pltpu.VMEM((SC_CHUNK, d), jnp.float32),
            pltpu.SemaphoreType.DMA,
        ],
    )
    def dispatch(x_hbm, dest_hbm, z_hbm, idx_v, rows_v, sem):
        wid = lax.axis_index("s") * SC_CORES + lax.axis_index("c")
        for c in range(iters):
            base = wid * (iters * SC_CHUNK) + c * SC_CHUNK
            pltpu.sync_copy(dest_hbm.at[pl.ds(base, SC_CHUNK)], idx_v)
            pltpu.sync_copy(x_hbm.at[pl.ds(base, SC_CHUNK)], rows_v)
            pltpu.async_copy(rows_v, z_hbm.at[idx_v], sem).wait()

    return dispatch


def _make_combine(bt, d):
    iters = bt // (NW * SC_CHUNK)
    mesh = plsc.VectorSubcoreMesh(core_axis_name="c", subcore_axis_name="s")

    @functools.partial(
        pl.kernel,
        out_type=jax.ShapeDtypeStruct((bt, d), jnp.float32),
        mesh=mesh,
        scratch_types=[
            pltpu.VMEM((SC_CHUNK,), jnp.int32),
            pltpu.VMEM((SC_CHUNK, d), jnp.float32),
            pltpu.SemaphoreType.DMA,
        ],
    )
    def combine(big_hbm, dest_hbm, y_hbm, idx_v, rows_v, sem):
        wid = lax.axis_index("s") * SC_CORES + lax.axis_index("c")
        for c in range(iters):
            base = wid * (iters * SC_CHUNK) + c * SC_CHUNK
            pltpu.sync_copy(dest_hbm.at[pl.ds(base, SC_CHUNK)], idx_v)
            pltpu.async_copy(big_hbm.at[idx_v], rows_v, sem).wait()
            pltpu.sync_copy(rows_v, y_hbm.at[pl.ds(base, SC_CHUNK)])

    return combine


def kernel(x, Wg, W1, b1, W2, b2):
    bb, tt, d = x.shape
    n = Wg.shape[1]
    hdim = W1.shape[2]
    bt = bb * tt
    cap = math.ceil(1.25 * (bt / n))
    nslot = n * cap
    zrows = nslot + bt
    hb = 2048
    nh = hdim // hb

    logits = jnp.einsum('btd,dn->btn', x, Wg)
    probs = jax.nn.softmax(logits, axis=-1)
    pf = probs.reshape(bt, n)
    top1_val = jnp.max(pf, axis=-1)
    top1_idx = jnp.argmax(pf, axis=-1).astype(jnp.int32)
    x_flat = x.reshape(bt, d)

    dest_row = pl.pallas_call(
        functools.partial(_rank_body, bt=bt, n=n, cap=cap, nslot=nslot),
        out_shape=jax.ShapeDtypeStruct((1, bt), jnp.int32),
    )(top1_val.reshape(1, bt), top1_idx.reshape(1, bt))
    dest = dest_row.reshape(bt)

    z = _make_dispatch(bt, d, zrows)(x_flat, dest)

    big = pl.pallas_call(
        _ffn_body,
        grid=(n, nh),
        in_specs=[
            pl.BlockSpec((cap, d), lambda e, h: (e, 0)),
            pl.BlockSpec((1, d, hb), lambda e, h: (e, 0, h)),
            pl.BlockSpec((1, 1, hb), lambda e, h: (e, 0, h)),
            pl.BlockSpec((1, hb, d), lambda e, h: (e, h, 0)),
            pl.BlockSpec((1, 1, d), lambda e, h: (e, 0, 0)),
        ],
        out_specs=pl.BlockSpec((cap, d), lambda e, h: (e, 0)),
        out_shape=jax.ShapeDtypeStruct((zrows, d), jnp.float32),
        input_output_aliases={0: 0},
    )(z, W1, b1.reshape(n, 1, hdim), W2, b2.reshape(n, 1, d))

    y = _make_combine(bt, d)(big, dest)
    return y.reshape(bb, tt, d)
